# Initial kernel scaffold; baseline (speedup 1.0000x reference)
import jax
import jax.numpy as jnp
from jax.experimental import pallas as pl


def kernel(x, code_book):
    raise NotImplementedError("write your pallas kernel here")


# speedup vs baseline: 2.9329x; 2.9329x over previous
import functools

import jax
import jax.numpy as jnp
from jax import lax
from jax.experimental import pallas as pl
from jax.experimental.pallas import tpu as pltpu

_KS = 512
_DIM = 64


def _pick_w(total):
    for w in (1792, 1024, 512, 256, 128, 64, 32, 16, 8):
        if total % w == 0:
            return w
    return total


def _vq_body(book_ref, x_ref, o_ref, *, w):
    book = book_ref[...]
    xt = x_ref[0]
    iota = lax.broadcasted_iota(jnp.int32, (_KS, w), 0)

    def depth(vt):
        ipT = lax.dot_general(book, vt, (((1,), (0,)), ((), ())),
                              preferred_element_type=jnp.float32)
        u = jnp.max(ipT, axis=0)
        c = jnp.min(jnp.where(ipT == u[None, :], iota, _KS), axis=0)
        oh = (iota == c[None, :]).astype(jnp.float32)
        comp = lax.dot_general(book, oh, (((0,), (0,)), ((), ())),
                               preferred_element_type=jnp.float32,
                               precision=lax.Precision.HIGHEST)
        return comp * u[None, :]

    comp1 = depth(xt)
    comp2 = depth(xt - comp1)
    o_ref[0] = comp1 + comp2


def kernel(x, code_book):
    n, dim, h, wd = x.shape
    total = h * wd
    w = _pick_w(total)
    xr = x.reshape(n, dim, total)
    grid = (n, total // w)
    out = pl.pallas_call(
        functools.partial(_vq_body, w=w),
        grid=grid,
        in_specs=[
            pl.BlockSpec((_KS, _DIM), lambda i, j: (0, 0)),
            pl.BlockSpec((1, dim, w), lambda i, j: (i, 0, j)),
        ],
        out_specs=pl.BlockSpec((1, dim, w), lambda i, j: (i, 0, j)),
        out_shape=jax.ShapeDtypeStruct((n, dim, total), jnp.float32),
        compiler_params=pltpu.CompilerParams(
            dimension_semantics=("parallel", "parallel"),
        ),
    )(code_book, xr)
    return out.reshape(n, dim, h, wd)


# speedup vs baseline: 4.9373x; 1.6834x over previous
import functools

import jax
import jax.numpy as jnp
from jax import lax
from jax.experimental import pallas as pl
from jax.experimental.pallas import tpu as pltpu

_KS = 512
_DIM = 64


def _pick_w(total):
    for w in (1792, 1024, 512, 256, 128, 64, 32, 16, 8):
        if total % w == 0:
            return w
    return total


def _vq_body(book_ref, bs_ref, x_ref, o_ref, *, w):
    book = book_ref[...]
    b_hi = bs_ref[0:_KS]
    b_lo = bs_ref[_KS:2 * _KS]
    b_lo2 = bs_ref[2 * _KS:3 * _KS]
    xt = x_ref[0]

    def depth(vt):
        ipT = lax.dot_general(book, vt, (((1,), (0,)), ((), ())),
                              preferred_element_type=jnp.float32)
        u = jnp.max(ipT, axis=0)
        oh = (ipT == u[None, :]).astype(jnp.bfloat16)
        dn = (((0,), (0,)), ((), ()))
        comp = (lax.dot_general(b_hi, oh, dn, preferred_element_type=jnp.float32)
                + lax.dot_general(b_lo, oh, dn, preferred_element_type=jnp.float32)
                + lax.dot_general(b_lo2, oh, dn, preferred_element_type=jnp.float32))
        return comp * u[None, :]

    comp1 = depth(xt)
    comp2 = depth(xt - comp1)
    o_ref[0] = comp1 + comp2


def kernel(x, code_book):
    n, dim, h, wd = x.shape
    total = h * wd
    w = _pick_w(total)
    xr = x.reshape(n, dim, total)
    b_hi = code_book.astype(jnp.bfloat16)
    r1 = code_book - b_hi.astype(jnp.float32)
    b_lo = r1.astype(jnp.bfloat16)
    b_lo2 = (r1 - b_lo.astype(jnp.float32)).astype(jnp.bfloat16)
    bsplit = jnp.concatenate([b_hi, b_lo, b_lo2], axis=0)
    grid = (n, total // w)
    out = pl.pallas_call(
        functools.partial(_vq_body, w=w),
        grid=grid,
        in_specs=[
            pl.BlockSpec((_KS, _DIM), lambda i, j: (0, 0)),
            pl.BlockSpec((3 * _KS, _DIM), lambda i, j: (0, 0)),
            pl.BlockSpec((1, dim, w), lambda i, j: (i, 0, j)),
        ],
        out_specs=pl.BlockSpec((1, dim, w), lambda i, j: (i, 0, j)),
        out_shape=jax.ShapeDtypeStruct((n, dim, total), jnp.float32),
        compiler_params=pltpu.CompilerParams(
            dimension_semantics=("parallel", "parallel"),
        ),
    )(code_book, bsplit, xr)
    return out.reshape(n, dim, h, wd)


# speedup vs baseline: 5.8545x; 1.1858x over previous
import functools

import jax
import jax.numpy as jnp
from jax import lax
from jax.experimental import pallas as pl
from jax.experimental.pallas import tpu as pltpu

_KS = 512
_DIM = 64


def _pick_w(total):
    for w in (3584, 1792, 1024, 512, 256, 128, 64, 32, 16, 8):
        if total % w == 0:
            return w
    return total


def _vq_body(book_ref, bs_ref, x_ref, o_ref, *, w):
    book = book_ref[...]
    b_hi = bs_ref[0:_KS]
    b_lo = bs_ref[_KS:2 * _KS]
    b_lo2 = bs_ref[2 * _KS:3 * _KS]
    xt = x_ref[0]

    dn = (((0,), (0,)), ((), ()))

    def depth(vt, exact):
        ipT = lax.dot_general(book, vt, (((1,), (0,)), ((), ())),
                              preferred_element_type=jnp.float32)
        u = jnp.max(ipT, axis=0)
        oh = (ipT == u[None, :]).astype(jnp.bfloat16)
        comp = lax.dot_general(b_hi, oh, dn, preferred_element_type=jnp.float32)
        if exact:
            comp = (comp
                    + lax.dot_general(b_lo, oh, dn, preferred_element_type=jnp.float32)
                    + lax.dot_general(b_lo2, oh, dn, preferred_element_type=jnp.float32))
        return comp * u[None, :]

    comp1 = depth(xt, True)
    comp2 = depth(xt - comp1, False)
    o_ref[0] = comp1 + comp2


def kernel(x, code_book):
    n, dim, h, wd = x.shape
    total = h * wd
    w = _pick_w(total)
    xr = x.reshape(n, dim, total)
    b_hi = code_book.astype(jnp.bfloat16)
    r1 = code_book - b_hi.astype(jnp.float32)
    b_lo = r1.astype(jnp.bfloat16)
    b_lo2 = (r1 - b_lo.astype(jnp.float32)).astype(jnp.bfloat16)
    bsplit = jnp.concatenate([b_hi, b_lo, b_lo2], axis=0)
    grid = (n, total // w)
    out = pl.pallas_call(
        functools.partial(_vq_body, w=w),
        grid=grid,
        in_specs=[
            pl.BlockSpec((_KS, _DIM), lambda i, j: (0, 0)),
            pl.BlockSpec((3 * _KS, _DIM), lambda i, j: (0, 0)),
            pl.BlockSpec((1, dim, w), lambda i, j: (i, 0, j)),
        ],
        out_specs=pl.BlockSpec((1, dim, w), lambda i, j: (i, 0, j)),
        out_shape=jax.ShapeDtypeStruct((n, dim, total), jnp.float32),
        compiler_params=pltpu.CompilerParams(
            dimension_semantics=("parallel", "parallel"),
        ),
    )(code_book, bsplit, xr)
    return out.reshape(n, dim, h, wd)


# speedup vs baseline: 5.8885x; 1.0058x over previous
import functools

import jax
import jax.numpy as jnp
from jax import lax
from jax.experimental import pallas as pl
from jax.experimental.pallas import tpu as pltpu

_KS = 512
_DIM = 64


def _pick_w(total):
    for w in (3584, 1792, 1024, 512, 256, 128, 64, 32, 16, 8):
        if total % w == 0:
            return w
    return total


def _vq_body(bs_ref, x_ref, o_ref, *, w):
    b_hi = bs_ref[0:_KS]
    b_lo = bs_ref[_KS:2 * _KS]
    b_lo2 = bs_ref[2 * _KS:3 * _KS]
    xt = x_ref[0]

    dn = (((0,), (0,)), ((), ()))

    def depth(vt, exact):
        ipT = lax.dot_general(b_hi, vt.astype(jnp.bfloat16),
                              (((1,), (0,)), ((), ())),
                              preferred_element_type=jnp.float32)
        u = jnp.max(ipT, axis=0)
        oh = (ipT == u[None, :]).astype(jnp.bfloat16)
        comp = lax.dot_general(b_hi, oh, dn, preferred_element_type=jnp.float32)
        if exact:
            comp = (comp
                    + lax.dot_general(b_lo, oh, dn, preferred_element_type=jnp.float32)
                    + lax.dot_general(b_lo2, oh, dn, preferred_element_type=jnp.float32))
        return comp * u[None, :]

    comp1 = depth(xt, True)
    comp2 = depth(xt - comp1, False)
    o_ref[0] = comp1 + comp2


def kernel(x, code_book):
    n, dim, h, wd = x.shape
    total = h * wd
    w = _pick_w(total)
    xr = x.reshape(n, dim, total)
    b_hi = code_book.astype(jnp.bfloat16)
    r1 = code_book - b_hi.astype(jnp.float32)
    b_lo = r1.astype(jnp.bfloat16)
    b_lo2 = (r1 - b_lo.astype(jnp.float32)).astype(jnp.bfloat16)
    bsplit = jnp.concatenate([b_hi, b_lo, b_lo2], axis=0)
    grid = (n, total // w)
    out = pl.pallas_call(
        functools.partial(_vq_body, w=w),
        grid=grid,
        in_specs=[
            pl.BlockSpec((3 * _KS, _DIM), lambda i, j: (0, 0)),
            pl.BlockSpec((1, dim, w), lambda i, j: (i, 0, j)),
        ],
        out_specs=pl.BlockSpec((1, dim, w), lambda i, j: (i, 0, j)),
        out_shape=jax.ShapeDtypeStruct((n, dim, total), jnp.float32),
        compiler_params=pltpu.CompilerParams(
            dimension_semantics=("parallel", "parallel"),
        ),
    )(bsplit, xr)
    return out.reshape(n, dim, h, wd)


# speedup vs baseline: 5.9364x; 1.0081x over previous
import functools

import jax
import jax.numpy as jnp
from jax import lax
from jax.experimental import pallas as pl
from jax.experimental.pallas import tpu as pltpu

_KS = 512
_DIM = 64


def _pick_w(total):
    for w in (7168, 3584, 1792, 1024, 512, 256, 128, 64, 32, 16, 8):
        if total % w == 0:
            return w
    return total


def _vq_body(bs_ref, x_ref, o_ref, *, w):
    b_hi = bs_ref[0:_KS]
    b_lo = bs_ref[_KS:2 * _KS]
    b_lo2 = bs_ref[2 * _KS:3 * _KS]
    xt = x_ref[0]

    dn = (((0,), (0,)), ((), ()))

    def depth(vt, exact):
        ipT = lax.dot_general(b_hi, vt.astype(jnp.bfloat16),
                              (((1,), (0,)), ((), ())),
                              preferred_element_type=jnp.float32)
        u = jnp.max(ipT, axis=0)
        oh = (ipT == u[None, :]).astype(jnp.bfloat16)
        comp = lax.dot_general(b_hi, oh, dn, preferred_element_type=jnp.float32)
        if exact:
            comp = (comp
                    + lax.dot_general(b_lo, oh, dn, preferred_element_type=jnp.float32)
                    + lax.dot_general(b_lo2, oh, dn, preferred_element_type=jnp.float32))
        return comp * u[None, :]

    comp1 = depth(xt, True)
    comp2 = depth(xt - comp1, False)
    o_ref[0] = comp1 + comp2


def kernel(x, code_book):
    n, dim, h, wd = x.shape
    total = h * wd
    w = _pick_w(total)
    xr = x.reshape(n, dim, total)
    b_hi = code_book.astype(jnp.bfloat16)
    r1 = code_book - b_hi.astype(jnp.float32)
    b_lo = r1.astype(jnp.bfloat16)
    b_lo2 = (r1 - b_lo.astype(jnp.float32)).astype(jnp.bfloat16)
    bsplit = jnp.concatenate([b_hi, b_lo, b_lo2], axis=0)
    grid = (n, total // w)
    out = pl.pallas_call(
        functools.partial(_vq_body, w=w),
        grid=grid,
        in_specs=[
            pl.BlockSpec((3 * _KS, _DIM), lambda i, j: (0, 0)),
            pl.BlockSpec((1, dim, w), lambda i, j: (i, 0, j)),
        ],
        out_specs=pl.BlockSpec((1, dim, w), lambda i, j: (i, 0, j)),
        out_shape=jax.ShapeDtypeStruct((n, dim, total), jnp.float32),
        compiler_params=pltpu.CompilerParams(
            dimension_semantics=("parallel", "parallel"),
        ),
    )(bsplit, xr)
    return out.reshape(n, dim, h, wd)
